# baseline (speedup 1.0000x reference)
import functools

import jax
import jax.numpy as jnp
from jax import lax
from jax.experimental import pallas as pl
from jax.experimental.pallas import tpu as pltpu
from jax.experimental.pallas import tpu_sc as plsc

NC = 2
NS = 16
K = 400


def _agg_body(npad, rpt, iters, dh,
              x2_hbm, src_hbm, dst_hbm,
              sum_hbm, cnt_hbm,
              idx0_v, idx1_v, rows0_v, rows1_v, ones_v, z16_v, pair_v,
              shared_sum, shared_cnt,
              semi0, semi1, semg0, semg1):
    cid = lax.axis_index("c")
    sid = lax.axis_index("s")
    rb = sid * rpt

    idx_v = [idx0_v, idx1_v]
    semi = [semi0, semi1]
    rows_v = [rows0_v, rows1_v]
    semg = [semg0, semg1]

    def idx_fetch(j, b):
        pltpu.make_async_copy(src_hbm.at[sid, j], idx_v[b].at[0], semi[b]).start()
        pltpu.make_async_copy(dst_hbm.at[sid, j], idx_v[b].at[1], semi[b]).start()

    def idx_wait(j, b):
        pltpu.make_async_copy(src_hbm.at[sid, j], idx_v[b].at[0], semi[b]).wait()
        pltpu.make_async_copy(dst_hbm.at[sid, j], idx_v[b].at[1], semi[b]).wait()

    def fix_src(b):
        def t(ti, carry):
            o = ti * 16
            v = idx_v[b][0, pl.ds(o, 16)]
            idx_v[b][0, pl.ds(o, 16)] = v + v + cid
            return carry

        lax.fori_loop(0, K // 16, t, 0)

    def gather(b):
        pltpu.make_async_copy(
            x2_hbm.at[idx_v[b].at[0]], rows_v[b], semg[b]).start()

    def gather_wait(b):
        pltpu.make_async_copy(
            x2_hbm.at[idx_v[b].at[0]], rows_v[b], semg[b]).wait()

    idx_fetch(0, 0)
    idx_fetch(1, 1)

    zv = jnp.zeros((16,), jnp.float32)
    ov = jnp.ones((16,), jnp.float32)

    def fill(r, carry):
        for c in range(dh // 16):
            rows0_v[r, pl.ds(c * 16, 16)] = zv
        z16_v[r, pl.ds(0, 16)] = zv
        ones_v[r, pl.ds(0, 16)] = ov
        return carry

    lax.fori_loop(0, K, fill, 0)
    off = 0
    while off < rpt:
        w = min(K, rpt - off)
        pltpu.sync_copy(rows0_v.at[pl.ds(0, w)],
                        shared_sum.at[pl.ds(rb + off, w)])
        pltpu.sync_copy(z16_v.at[pl.ds(0, w)],
                        shared_cnt.at[pl.ds(rb + off, w)])
        off += w

    idx_wait(0, 0)
    fix_src(0)
    gather(0)
    plsc.subcore_barrier()

    def step(j, b, b1):
        @pl.when(j + 1 < iters)
        def _():
            idx_wait(j + 1, b1)
            fix_src(b1)
            gather(b1)

        gather_wait(b)
        pltpu.sync_copy(rows_v[b], shared_sum.at[idx_v[b].at[1]], add=True)

        @pl.when(lax.rem(j, 2) == cid)
        def _():
            pltpu.sync_copy(ones_v, shared_cnt.at[idx_v[b].at[1]], add=True)

        @pl.when(j + 2 < iters)
        def _():
            idx_fetch(j + 2, b)

    def body2(i, carry):
        j0 = 2 * i
        step(j0, 0, 1)
        step(j0 + 1, 1, 0)
        return carry

    lax.fori_loop(0, iters // 2, body2, 0)

    plsc.subcore_barrier()
    off = 0
    while off < rpt:
        w = min(K // 2, rpt - off)
        pltpu.sync_copy(shared_sum.at[pl.ds(rb + off, w)],
                        rows0_v.at[pl.ds(0, w)])

        def rp(r, carry):
            for c in range(dh // 16):
                pair_v[r, pl.ds(c * 16, 16)] = rows0_v[2 * r, pl.ds(c * 16, 16)]
                pair_v[r, pl.ds(dh + c * 16, 16)] = \
                    rows0_v[2 * r + 1, pl.ds(c * 16, 16)]
            return carry

        lax.fori_loop(0, w // 2, rp, 0)
        pltpu.sync_copy(pair_v.at[pl.ds(0, w // 2)],
                        sum_hbm.at[cid, pl.ds((rb + off) // 2, w // 2)])
        off += w

    off = 0
    while off < rpt:
        w = min(K // 2, rpt - off)
        pltpu.sync_copy(shared_cnt.at[pl.ds(rb + off, w)],
                        ones_v.at[pl.ds(0, w)])

        def rc(r, carry):
            for c in range(4):
                pair_v[r, pl.ds(c * 16, 16)] = ones_v[2 * r, pl.ds(0, 16)]
                pair_v[r, pl.ds(dh + c * 16, 16)] = \
                    ones_v[2 * r + 1, pl.ds(0, 16)]
            return carry

        lax.fori_loop(0, w // 2, rc, 0)
        pltpu.sync_copy(pair_v.at[pl.ds(0, w // 2)],
                        cnt_hbm.at[cid, pl.ds((rb + off) // 2, w // 2)])
        off += w


def _aggregate(x2, src, dst):
    n2, dh = x2.shape
    n = n2 // 2
    e = src.shape[0]
    npad = ((n + 1 + 127) // 128) * 128
    rpt = npad // NS
    e_pad = ((e + NS * K - 1) // (NS * K)) * (NS * K)
    if e_pad % (2 * NS * K):
        e_pad += NS * K
    if e_pad != e:
        src = jnp.concatenate([src, jnp.zeros((e_pad - e,), jnp.int32)])
        dst = jnp.concatenate([dst, jnp.full((e_pad - e,), n, jnp.int32)])
    ept = e_pad // NS
    iters = ept // K

    src3 = src.reshape(NS, iters, K)
    dst3 = dst.reshape(NS, iters, K)

    mesh = plsc.VectorSubcoreMesh(core_axis_name="c", subcore_axis_name="s",
                                  num_cores=NC, num_subcores=NS)
    body = functools.partial(_agg_body, npad, rpt, iters, dh)
    return pl.kernel(
        body,
        out_type=(jax.ShapeDtypeStruct((NC, npad // 2, 2 * dh), jnp.float32),
                  jax.ShapeDtypeStruct((NC, npad // 2, 2 * dh), jnp.float32)),
        mesh=mesh,
        scratch_types=[
            pltpu.VMEM((2, K), jnp.int32),
            pltpu.VMEM((2, K), jnp.int32),
            pltpu.VMEM((K, dh), jnp.float32),
            pltpu.VMEM((K, dh), jnp.float32),
            pltpu.VMEM((K, 16), jnp.float32),
            pltpu.VMEM((K, 16), jnp.float32),
            pltpu.VMEM((K // 4, 128), jnp.float32),
            pltpu.VMEM_SHARED((npad, dh), jnp.float32),
            pltpu.VMEM_SHARED((npad, 16), jnp.float32),
            pltpu.SemaphoreType.DMA,
            pltpu.SemaphoreType.DMA,
            pltpu.SemaphoreType.DMA,
            pltpu.SemaphoreType.DMA,
        ],
        compiler_params=pltpu.CompilerParams(use_tc_tiling_on_sc=False),
    )(x2, src3, dst3)


def _dense_body(sums_ref, cnts_ref, x_ref, wll_ref, wlr_ref, wr_ref,
                wo_ref, bl_ref, bo_ref, g1_ref, b1_ref, g2_ref, b2_ref,
                out_ref):
    c = cnts_ref[0, 0] + cnts_ref[1, 0]
    inv = 1.0 / jnp.maximum(c, 1.0)
    ml = sums_ref[0, 0] * inv
    mr = sums_ref[1, 0] * inv
    dn = (((1,), (1,)), ((), ()))
    for p in range(2):
        h = (lax.dot_general(ml[:, p, :], wll_ref[...], dn,
                             preferred_element_type=jnp.float32)
             + lax.dot_general(mr[:, p, :], wlr_ref[...], dn,
                               preferred_element_type=jnp.float32)
             + bl_ref[...]
             + lax.dot_general(x_ref[:, p, :], wr_ref[...], dn,
                               preferred_element_type=jnp.float32))
        mu = jnp.mean(h, axis=-1, keepdims=True)
        var = jnp.mean((h - mu) ** 2, axis=-1, keepdims=True)
        h = (h - mu) * lax.rsqrt(var + 1e-5) * g1_ref[...] + b1_ref[...]
        h = 0.5 * h * (1.0 + lax.erf(h * 0.7071067811865476))
        o = (lax.dot_general(h, wo_ref[...], dn,
                             preferred_element_type=jnp.float32) + bo_ref[...])
        mu = jnp.mean(o, axis=-1, keepdims=True)
        var = jnp.mean((o - mu) ** 2, axis=-1, keepdims=True)
        out_ref[:, p, :] = (o - mu) * lax.rsqrt(var + 1e-5) * g2_ref[...] \
            + b2_ref[...]


def _dense(sums, cnts, x, W_l, b_l, W_r, ln1_g, ln1_b, W_out, b_out, ln2_g, ln2_b):
    n, d = x.shape
    dh = d // 2
    npad = sums.shape[1] * 2
    g = 2
    srows = (npad // 2) // g
    sums5 = sums.reshape(NC, g, srows, 2, dh)
    cnts5 = cnts.reshape(NC, g, srows, 2, dh)
    x3 = x.reshape(n // 2, 2, d)
    grid = (g,)
    full = lambda i: (0, 0)
    out = pl.pallas_call(
        _dense_body,
        grid=grid,
        in_specs=[
            pl.BlockSpec((NC, 1, srows, 2, dh), lambda i: (0, i, 0, 0, 0)),
            pl.BlockSpec((NC, 1, srows, 2, dh), lambda i: (0, i, 0, 0, 0)),
            pl.BlockSpec((srows, 2, d), lambda i: (i, 0, 0)),
            pl.BlockSpec((d, dh), full),
            pl.BlockSpec((d, dh), full),
            pl.BlockSpec((d, d), full),
            pl.BlockSpec((d, d), full),
            pl.BlockSpec((1, d), full),
            pl.BlockSpec((1, d), full),
            pl.BlockSpec((1, d), full),
            pl.BlockSpec((1, d), full),
            pl.BlockSpec((1, d), full),
            pl.BlockSpec((1, d), full),
        ],
        out_specs=pl.BlockSpec((srows, 2, d), lambda i: (i, 0, 0)),
        out_shape=jax.ShapeDtypeStruct((n // 2, 2, d), jnp.float32),
    )(sums5, cnts5, x3, W_l[:, :dh], W_l[:, dh:], W_r, W_out,
      b_l.reshape(1, d), b_out.reshape(1, d),
      ln1_g.reshape(1, d), ln1_b.reshape(1, d),
      ln2_g.reshape(1, d), ln2_b.reshape(1, d))
    return out.reshape(n, d)


def kernel(x, edge_index, W_l, b_l, W_r, ln1_g, ln1_b, W_out, b_out, ln2_g, ln2_b):
    src = edge_index[0].astype(jnp.int32)
    dst = edge_index[1].astype(jnp.int32)
    dh = x.shape[1] // 2
    x2 = x.reshape(-1, dh)
    sums, cnts = _aggregate(x2, src, dst)
    return _dense(sums, cnts, x, W_l, b_l, W_r, ln1_g, ln1_b,
                  W_out, b_out, ln2_g, ln2_b)
